# Initial kernel scaffold; baseline (speedup 1.0000x reference)
import jax
import jax.numpy as jnp
from jax.experimental import pallas as pl


def kernel(x_embed, prompt, W, b):
    raise NotImplementedError("write your pallas kernel here")


# speedup vs baseline: 1.0149x; 1.0149x over previous
import functools

import jax
import jax.numpy as jnp
from jax import lax
from jax.experimental import pallas as pl
from jax.experimental.pallas import tpu as pltpu


def _fused_body(nsb, x_ref, w_ref, b_ref, pr_ref, logits_ref, gath_ref, acc_ref):
    bi = pl.program_id(0)
    ci = pl.program_id(1)
    m = jnp.max(x_ref[...], axis=0, keepdims=True)

    @pl.when(ci == 0)
    def _init():
        acc_ref[pl.ds(bi, 1), :] = m

    @pl.when(ci > 0)
    def _acc():
        acc_ref[pl.ds(bi, 1), :] = jnp.maximum(acc_ref[pl.ds(bi, 1), :], m)

    B = logits_ref.shape[0]
    TWO = gath_ref.shape[1]

    @pl.when((bi == B - 1) & (ci == nsb - 1))
    def _final():
        xmax = acc_ref[...]
        logits = lax.dot_general(
            xmax, w_ref[...], (((1,), (1,)), ((), ())),
            preferred_element_type=jnp.float32,
        ) + b_ref[...]
        logits_ref[...] = logits
        idx = jnp.argmax(logits, axis=1).astype(jnp.int32)
        for bb in range(B):
            ib = idx[bb]
            for k in range(TWO):
                gath_ref[bb, k] = pr_ref[k, pl.ds(ib, 1)][0]


def kernel(x_embed, prompt, W, b):
    B, S, E = x_embed.shape
    NL, TWO, P, L, H, D = prompt.shape
    S_BLK = 512
    nsb = S // S_BLK
    x2 = x_embed.reshape(B * S, E)
    pr = prompt.reshape(TWO, P, L * H, D)
    logits, gath = pl.pallas_call(
        functools.partial(_fused_body, nsb),
        grid=(B, nsb),
        in_specs=[
            pl.BlockSpec((S_BLK, E), lambda bi, ci: (bi * nsb + ci, 0)),
            pl.BlockSpec((P, E), lambda bi, ci: (0, 0)),
            pl.BlockSpec((1, P), lambda bi, ci: (0, 0)),
            pl.BlockSpec((TWO, P, L * H, D), lambda bi, ci: (0, 0, 0, 0)),
        ],
        out_specs=[
            pl.BlockSpec((B, P), lambda bi, ci: (0, 0)),
            pl.BlockSpec((B, TWO, L * H, D), lambda bi, ci: (0, 0, 0, 0)),
        ],
        out_shape=[
            jax.ShapeDtypeStruct((B, P), jnp.float32),
            jax.ShapeDtypeStruct((B, TWO, L * H, D), jnp.float32),
        ],
        scratch_shapes=[pltpu.VMEM((B, E), jnp.float32)],
        compiler_params=pltpu.CompilerParams(
            dimension_semantics=("arbitrary", "arbitrary")),
    )(x2, W, b.reshape(1, P), pr)
    e_prompt = gath.reshape(B, TWO, L, H, D)[None]
    return logits, e_prompt
